# baseline (speedup 1.0000x reference)
import jax
import jax.numpy as jnp
from jax import lax
from jax.experimental import pallas as pl
from jax.experimental.pallas import tpu as pltpu
from jax.experimental.pallas import tpu_sc as plsc

D = 64
SEQ = 200
NB = 1024

_info = plsc.get_sparse_core_info()
_NC, _NS, _L = _info.num_cores, _info.num_subcores, _info.num_lanes
NW = _NC * _NS
TOTAL = NB * SEQ
PER_W = TOTAL // NW
CHUNK = 128
NCHUNK = PER_W // CHUNK


def _sc_body(x_hbm, cos_hbm, sin_hbm, table_hbm, out_hbm,
             idx_v, rows_v, cos_v, sin_v, sem):
    wid = lax.axis_index("s") * _NC + lax.axis_index("c")
    base = wid * PER_W
    pltpu.sync_copy(x_hbm.at[wid], idx_v)
    pltpu.sync_copy(cos_hbm, cos_v)
    pltpu.sync_copy(sin_hbm, sin_v)
    iota = lax.iota(jnp.int32, _L)

    def chunk_body(c, carry):
        pltpu.async_copy(table_hbm.at[idx_v.at[c]], rows_v, sem).wait()

        def row_body(r, carry2):
            s = lax.rem(c * CHUNK + r, SEQ)
            rsplat = jnp.full((_L,), r, jnp.int32)
            for k in range(2):
                col1 = 2 * iota + (2 * _L * k)
                col2 = col1 + 1
                x1 = plsc.load_gather(rows_v, [rsplat, col1])
                x2 = plsc.load_gather(rows_v, [rsplat, col2])
                cc = cos_v[s, pl.ds(_L * k, _L)]
                ss = sin_v[s, pl.ds(_L * k, _L)]
                plsc.store_scatter(rows_v, [rsplat, col1], x1 * cc - x2 * ss)
                plsc.store_scatter(rows_v, [rsplat, col2], x1 * ss + x2 * cc)
            return carry2

        lax.fori_loop(0, CHUNK, row_body, 0)
        pltpu.sync_copy(rows_v, out_hbm.at[pl.ds(base + c * CHUNK, CHUNK)])
        return carry

    lax.fori_loop(0, NCHUNK, chunk_body, 0)


def kernel(x, table):
    b, s = x.shape
    x_r = x.reshape(NW, NCHUNK, CHUNK).astype(jnp.int32)
    inv_freq = 1.0 / (10000.0 ** (jnp.arange(0, D, 2, dtype=jnp.float32) / D))
    pos = jnp.arange(SEQ, dtype=jnp.float32)
    freqs = pos[:, None] * inv_freq[None, :]
    cos_t = jnp.cos(freqs)
    sin_t = jnp.sin(freqs)

    mesh = plsc.VectorSubcoreMesh(core_axis_name="c", subcore_axis_name="s")
    f = pl.kernel(
        _sc_body,
        out_type=jax.ShapeDtypeStruct((TOTAL, D), jnp.float32),
        mesh=mesh,
        compiler_params=pltpu.CompilerParams(needs_layout_passes=False,
                                             use_tc_tiling_on_sc=False),
        scratch_types=[
            pltpu.VMEM((NCHUNK, CHUNK), jnp.int32),
            pltpu.VMEM((CHUNK, D), jnp.float32),
            pltpu.VMEM((SEQ, D // 2), jnp.float32),
            pltpu.VMEM((SEQ, D // 2), jnp.float32),
            pltpu.SemaphoreType.DMA,
        ],
    )
    out = f(x_r, cos_t, sin_t, table)
    return out.reshape(b, s, D)
